# Initial kernel scaffold; baseline (speedup 1.0000x reference)
import functools

import jax
import jax.numpy as jnp
from jax import lax
from jax.experimental import pallas as pl
from jax.experimental.pallas import tpu as pltpu
from jax.experimental.pallas import tpu_sc as plsc

NC = 2
NS = 16
NW = NC * NS
CH = 128


def _sc_aggregate(nj, n_pad, rows_per_tile, with_counts):
  out_type = [jax.ShapeDtypeStruct((NC, n_pad, 16), jnp.float32)]
  if with_counts:
    out_type.append(jax.ShapeDtypeStruct((NC, n_pad, 16), jnp.float32))

  scratch = [
      pltpu.VMEM((nj, CH), jnp.int32),
      pltpu.VMEM((nj, CH), jnp.int32),
      pltpu.VMEM((2, CH, 16), jnp.float32),
      pltpu.VMEM((CH, 16), jnp.float32),
      pltpu.VMEM_SHARED((n_pad, 16), jnp.float32),
      pltpu.VMEM_SHARED((n_pad, 16), jnp.float32),
      pltpu.SemaphoreType.DMA,
  ]

  def body(src3, dst3, table, zeros, ones, *rest):
    if with_counts:
      acc_out, cnt_out = rest[0], rest[1]
      srcb, dstb, rows2, onesb, acc_sh, cnt_sh, gsem = rest[2:]
    else:
      acc_out = rest[0]
      srcb, dstb, rows2, onesb, acc_sh, cnt_sh, gsem = rest[1:]
    cid = lax.axis_index("c")
    sid = lax.axis_index("s")
    wid = cid * NS + sid
    rslice = pl.ds(sid * rows_per_tile, rows_per_tile)

    pltpu.sync_copy(zeros, acc_sh.at[rslice])
    if with_counts:
      pltpu.sync_copy(zeros, cnt_sh.at[rslice])
      pltpu.sync_copy(ones, onesb)
    pltpu.sync_copy(src3.at[wid], srcb)
    pltpu.sync_copy(dst3.at[wid], dstb)
    plsc.subcore_barrier()

    pltpu.async_copy(table.at[srcb.at[0]], rows2.at[0], gsem)

    def step(j, carry):
      nxt = j + 1

      @pl.when(nxt < nj)
      def _():
        pltpu.async_copy(table.at[srcb.at[nxt]], rows2.at[nxt % 2], gsem)

      pltpu.make_async_copy(table.at[srcb.at[j]], rows2.at[j % 2], gsem).wait()
      pltpu.sync_copy(rows2.at[j % 2], acc_sh.at[dstb.at[j]], add=True)
      if with_counts:
        pltpu.sync_copy(onesb, cnt_sh.at[dstb.at[j]], add=True)
      return carry

    lax.fori_loop(0, nj, step, 0)
    plsc.subcore_barrier()

    pltpu.sync_copy(acc_sh.at[rslice], acc_out.at[cid, rslice])
    if with_counts:
      pltpu.sync_copy(cnt_sh.at[rslice], cnt_out.at[cid, rslice])

  mesh = plsc.VectorSubcoreMesh(core_axis_name="c", subcore_axis_name="s")
  return pl.kernel(body, out_type=out_type, mesh=mesh, scratch_types=scratch)


def _mm_kernel(x_ref, w_ref, o_ref):
  o_ref[...] = jnp.dot(x_ref[...], w_ref[...],
                       preferred_element_type=jnp.float32)


def _h_kernel(n, acc_ref, cnt_ref, yz_ref, b0l_ref, g_ref, b_ref, o_ref):
  acc = acc_ref[0, :n, :] + acc_ref[1, :n, :]
  cnt = cnt_ref[0, :n, :] + cnt_ref[1, :n, :]
  agg = acc / jnp.maximum(cnt, 1.0)
  pre = agg + yz_ref[:, 16:32] + b0l_ref[...]
  hr = jnp.maximum(pre, 0.0)
  mu = jnp.mean(hr, axis=1, keepdims=True)
  var = jnp.mean((hr - mu) ** 2, axis=1, keepdims=True)
  o_ref[...] = (hr - mu) / jnp.sqrt(var + 1e-5) * g_ref[...] + b_ref[...]


def _out_kernel(n, acc_ref, cnt_ref, h_ref, w_ref, b_ref, o_ref):
  acc = acc_ref[0, :n, :] + acc_ref[1, :n, :]
  cnt = cnt_ref[0, :n, :] + cnt_ref[1, :n, :]
  agg = acc / jnp.maximum(cnt, 1.0)
  feat = jnp.concatenate([agg, h_ref[...]], axis=1)
  o_ref[...] = jnp.dot(feat, w_ref[...],
                       preferred_element_type=jnp.float32) + b_ref[...]


def kernel(x, edge_index, W0l, b0l, W0r, ln_g, ln_b, W1l, b1l, W1r):
  n, d_in = x.shape
  e = edge_index.shape[1]
  d_hid = W0l.shape[1]
  d_out = W1l.shape[1]

  nj = -(-e // (NW * CH))
  e_pad = NW * nj * CH
  rows_per_tile = -(-(n + 8) // NS)
  n_pad = NS * rows_per_tile

  src = edge_index[0]
  dst = edge_index[1]
  pad = e_pad - e
  src3 = jnp.concatenate([src, jnp.zeros((pad,), jnp.int32)]).reshape(NW, nj, CH)
  dst3 = jnp.concatenate([dst, jnp.full((pad,), n, jnp.int32)]).reshape(NW, nj, CH)
  zeros = jnp.zeros((rows_per_tile, 16), jnp.float32)
  ones = jnp.ones((CH, 16), jnp.float32)

  wcat0 = jnp.concatenate([W0l, W0r], axis=1)
  y0z0 = pl.pallas_call(
      _mm_kernel,
      out_shape=jax.ShapeDtypeStruct((n, 2 * d_hid), jnp.float32),
  )(x, wcat0)
  y0 = y0z0[:, :d_hid]

  agg_fn = _sc_aggregate(nj, n_pad, rows_per_tile, with_counts=True)
  acc0, cnt = agg_fn(src3, dst3, y0, zeros, ones)

  h = pl.pallas_call(
      functools.partial(_h_kernel, n),
      out_shape=jax.ShapeDtypeStruct((n, d_hid), jnp.float32),
  )(acc0, cnt, y0z0, b0l.reshape(1, -1), ln_g.reshape(1, -1),
    ln_b.reshape(1, -1))

  agg_fn2 = _sc_aggregate(nj, n_pad, rows_per_tile, with_counts=False)
  (acc1,) = agg_fn2(src3, dst3, h, zeros, ones)

  wcat1 = jnp.concatenate([W1l, W1r], axis=0)
  out = pl.pallas_call(
      functools.partial(_out_kernel, n),
      out_shape=jax.ShapeDtypeStruct((n, d_out), jnp.float32),
  )(acc1, cnt, h, wcat1, b1l.reshape(1, -1))
  return out


# speedup vs baseline: 16.5140x; 16.5140x over previous
import functools

import jax
import jax.numpy as jnp
from jax import lax
from jax.experimental import pallas as pl
from jax.experimental.pallas import tpu as pltpu
from jax.experimental.pallas import tpu_sc as plsc

NC = 2
NS = 16
NW = NC * NS
CH = 128


def _sc_aggregate(nj, n_pad, rows_per_tile, with_counts):
  out_type = [jax.ShapeDtypeStruct((NC, n_pad, 16), jnp.float32)]
  if with_counts:
    out_type.append(jax.ShapeDtypeStruct((NC, n_pad, 16), jnp.float32))

  scratch = [
      pltpu.VMEM((nj, CH), jnp.int32),
      pltpu.VMEM((nj, CH), jnp.int32),
      pltpu.VMEM((2, CH, 16), jnp.float32),
      pltpu.VMEM((CH, 16), jnp.float32),
      pltpu.VMEM_SHARED((n_pad, 16), jnp.float32),
      pltpu.VMEM_SHARED((n_pad, 16), jnp.float32),
      pltpu.SemaphoreType.DMA,
  ]

  def body(src3, dst3, table, zeros, ones, *rest):
    if with_counts:
      acc_out, cnt_out = rest[0], rest[1]
      srcb, dstb, rows2, onesb, acc_sh, cnt_sh, gsem = rest[2:]
    else:
      acc_out = rest[0]
      srcb, dstb, rows2, onesb, acc_sh, cnt_sh, gsem = rest[1:]
    cid = lax.axis_index("c")
    sid = lax.axis_index("s")
    wid = cid * NS + sid
    rslice = pl.ds(sid * rows_per_tile, rows_per_tile)

    pltpu.sync_copy(zeros, acc_sh.at[rslice])
    if with_counts:
      pltpu.sync_copy(zeros, cnt_sh.at[rslice])
      pltpu.sync_copy(ones, onesb)
    pltpu.sync_copy(src3.at[wid], srcb)
    pltpu.sync_copy(dst3.at[wid], dstb)
    plsc.subcore_barrier()

    pltpu.async_copy(table.at[srcb.at[0]], rows2.at[0], gsem)

    def step(j, carry):
      nxt = j + 1

      @pl.when(nxt < nj)
      def _():
        pltpu.async_copy(table.at[srcb.at[nxt]], rows2.at[nxt % 2], gsem)

      pltpu.make_async_copy(table.at[srcb.at[j]], rows2.at[j % 2], gsem).wait()
      pltpu.sync_copy(rows2.at[j % 2], acc_sh.at[dstb.at[j]], add=True)
      if with_counts:
        pltpu.sync_copy(onesb, cnt_sh.at[dstb.at[j]], add=True)
      return carry

    lax.fori_loop(0, nj, step, 0)
    plsc.subcore_barrier()

    pltpu.sync_copy(acc_sh.at[rslice], acc_out.at[cid, rslice])
    if with_counts:
      pltpu.sync_copy(cnt_sh.at[rslice], cnt_out.at[cid, rslice])

  mesh = plsc.VectorSubcoreMesh(core_axis_name="c", subcore_axis_name="s")
  return pl.kernel(body, out_type=out_type, mesh=mesh, scratch_types=scratch,
                   compiler_params=pltpu.CompilerParams(
                       use_tc_tiling_on_sc=False))


def _mm_kernel(x_ref, w_ref, o_ref):
  o_ref[...] = jnp.dot(x_ref[...], w_ref[...],
                       preferred_element_type=jnp.float32)


def _h_kernel(n, acc_ref, cnt_ref, yz_ref, b0l_ref, g_ref, b_ref, o_ref):
  acc = acc_ref[0, :n, :] + acc_ref[1, :n, :]
  cnt = cnt_ref[0, :n, :] + cnt_ref[1, :n, :]
  agg = acc / jnp.maximum(cnt, 1.0)
  pre = agg + yz_ref[:, 16:32] + b0l_ref[...]
  hr = jnp.maximum(pre, 0.0)
  mu = jnp.mean(hr, axis=1, keepdims=True)
  var = jnp.mean((hr - mu) ** 2, axis=1, keepdims=True)
  o_ref[...] = (hr - mu) / jnp.sqrt(var + 1e-5) * g_ref[...] + b_ref[...]


def _out_kernel(n, acc_ref, cnt_ref, h_ref, w_ref, b_ref, o_ref):
  acc = acc_ref[0, :n, :] + acc_ref[1, :n, :]
  cnt = cnt_ref[0, :n, :] + cnt_ref[1, :n, :]
  agg = acc / jnp.maximum(cnt, 1.0)
  feat = jnp.concatenate([agg, h_ref[...]], axis=1)
  o_ref[...] = jnp.dot(feat, w_ref[...],
                       preferred_element_type=jnp.float32) + b_ref[...]


def kernel(x, edge_index, W0l, b0l, W0r, ln_g, ln_b, W1l, b1l, W1r):
  n, d_in = x.shape
  e = edge_index.shape[1]
  d_hid = W0l.shape[1]
  d_out = W1l.shape[1]

  nj = -(-e // (NW * CH))
  e_pad = NW * nj * CH
  rows_per_tile = (-(-(n + 8) // NS) + 7) // 8 * 8
  n_pad = NS * rows_per_tile

  src = edge_index[0]
  dst = edge_index[1]
  pad = e_pad - e
  src3 = jnp.concatenate([src, jnp.zeros((pad,), jnp.int32)]).reshape(NW, nj, CH)
  dst3 = jnp.concatenate([dst, jnp.full((pad,), n, jnp.int32)]).reshape(NW, nj, CH)
  zeros = jnp.zeros((rows_per_tile, 16), jnp.float32)
  ones = jnp.ones((CH, 16), jnp.float32)

  wcat0 = jnp.concatenate([W0l, W0r], axis=1)
  y0z0 = pl.pallas_call(
      _mm_kernel,
      out_shape=jax.ShapeDtypeStruct((n, 2 * d_hid), jnp.float32),
  )(x, wcat0)
  y0 = y0z0[:, :d_hid]

  agg_fn = _sc_aggregate(nj, n_pad, rows_per_tile, with_counts=True)
  acc0, cnt = agg_fn(src3, dst3, y0, zeros, ones)

  h = pl.pallas_call(
      functools.partial(_h_kernel, n),
      out_shape=jax.ShapeDtypeStruct((n, d_hid), jnp.float32),
  )(acc0, cnt, y0z0, b0l.reshape(1, -1), ln_g.reshape(1, -1),
    ln_b.reshape(1, -1))

  agg_fn2 = _sc_aggregate(nj, n_pad, rows_per_tile, with_counts=False)
  (acc1,) = agg_fn2(src3, dst3, h, zeros, ones)

  wcat1 = jnp.concatenate([W1l, W1r], axis=0)
  out = pl.pallas_call(
      functools.partial(_out_kernel, n),
      out_shape=jax.ShapeDtypeStruct((n, d_out), jnp.float32),
  )(acc1, cnt, h, wcat1, b1l.reshape(1, -1))
  return out
